# Initial kernel scaffold; baseline (speedup 1.0000x reference)
import jax
import jax.numpy as jnp
from jax.experimental import pallas as pl


def kernel(z_e, embed_weight):
    raise NotImplementedError("write your pallas kernel here")


# speedup vs baseline: 2.5949x; 2.5949x over previous
import jax
import jax.numpy as jnp
from jax.experimental import pallas as pl
from jax.experimental.pallas import tpu as pltpu


def _vq_kernel(x_ref, e_ref, zq_ref, ind_ref, dsum_ref):
    x = x_ref[0]
    e = e_ref[...]
    esq = jnp.sum(e * e, axis=1, keepdims=True)
    s = 2.0 * jax.lax.dot_general(
        e, x, (((1,), (0,)), ((), ())),
        preferred_element_type=jnp.float32) - esq
    ind = jnp.argmax(s, axis=0).reshape(1, -1)
    oh = (jax.lax.broadcasted_iota(jnp.int32, s.shape, 0) == ind
          ).astype(jnp.float32)
    zq = jax.lax.dot_general(
        e, oh, (((0,), (0,)), ((), ())),
        preferred_element_type=jnp.float32)
    zq_ref[0] = zq
    ind_ref[0] = ind
    dsum_ref[0] = jnp.sum((zq - x) ** 2).reshape(1, 1)


def kernel(z_e, embed_weight):
    B, C, H, W = z_e.shape
    K = embed_weight.shape[0]
    P = H * W
    x = z_e.reshape(B, C, P)
    zq, ind3, dsums = pl.pallas_call(
        _vq_kernel,
        grid=(B,),
        in_specs=[
            pl.BlockSpec((1, C, P), lambda b: (b, 0, 0)),
            pl.BlockSpec((K, C), lambda b: (0, 0)),
        ],
        out_specs=[
            pl.BlockSpec((1, C, P), lambda b: (b, 0, 0)),
            pl.BlockSpec((1, 1, P), lambda b: (b, 0, 0)),
            pl.BlockSpec((1, 1, 1), lambda b: (b, 0, 0)),
        ],
        out_shape=[
            jax.ShapeDtypeStruct((B, C, P), jnp.float32),
            jax.ShapeDtypeStruct((B, 1, P), jnp.int32),
            jax.ShapeDtypeStruct((B, 1, 1), jnp.float32),
        ],
        compiler_params=pltpu.CompilerParams(
            dimension_semantics=("parallel",)),
    )(x, embed_weight)
    z_q_out = zq.reshape(B, C, H, W)
    ind = ind3.reshape(B, H, W)
    diff = jnp.sum(dsums) * (12.5 / (B * C * H * W))
    return (z_q_out, diff, ind)


# speedup vs baseline: 2.8847x; 1.1117x over previous
import jax
import jax.numpy as jnp
from jax.experimental import pallas as pl
from jax.experimental.pallas import tpu as pltpu

_BB = 2


def _vq_kernel(x_ref, e_ref, zq_ref, ind_ref, dsum_ref):
    e = e_ref[...]
    esqh = 0.5 * jnp.sum(e * e, axis=1, keepdims=True)
    for i in range(_BB):
        x = x_ref[i]
        s = jax.lax.dot_general(
            e, x, (((1,), (0,)), ((), ())),
            preferred_element_type=jnp.float32) - esqh
        ind = jnp.argmax(s, axis=0).reshape(1, -1)
        oh = (jax.lax.broadcasted_iota(jnp.int32, s.shape, 0) == ind
              ).astype(jnp.float32)
        zq = jax.lax.dot_general(
            e, oh, (((0,), (0,)), ((), ())),
            preferred_element_type=jnp.float32)
        zq_ref[i] = zq
        ind_ref[i] = ind
        dsum_ref[i] = jnp.sum((zq - x) ** 2).reshape(1, 1)


def kernel(z_e, embed_weight):
    B, C, H, W = z_e.shape
    K = embed_weight.shape[0]
    P = H * W
    x = z_e.reshape(B, C, P)
    zq, ind3, dsums = pl.pallas_call(
        _vq_kernel,
        grid=(B // _BB,),
        in_specs=[
            pl.BlockSpec((_BB, C, P), lambda b: (b, 0, 0)),
            pl.BlockSpec((K, C), lambda b: (0, 0)),
        ],
        out_specs=[
            pl.BlockSpec((_BB, C, P), lambda b: (b, 0, 0)),
            pl.BlockSpec((_BB, 1, P), lambda b: (b, 0, 0)),
            pl.BlockSpec((_BB, 1, 1), lambda b: (b, 0, 0)),
        ],
        out_shape=[
            jax.ShapeDtypeStruct((B, C, P), jnp.float32),
            jax.ShapeDtypeStruct((B, 1, P), jnp.int32),
            jax.ShapeDtypeStruct((B, 1, 1), jnp.float32),
        ],
        compiler_params=pltpu.CompilerParams(
            dimension_semantics=("parallel",)),
    )(x, embed_weight)
    z_q_out = zq.reshape(B, C, H, W)
    ind = ind3.reshape(B, H, W)
    diff = jnp.sum(dsums) * (12.5 / (B * C * H * W))
    return (z_q_out, diff, ind)


# speedup vs baseline: 2.9675x; 1.0287x over previous
import jax
import jax.numpy as jnp
from jax.experimental import pallas as pl
from jax.experimental.pallas import tpu as pltpu

_BB = 4


def _vq_kernel(x_ref, e_ref, zq_ref, ind_ref, dsum_ref):
    e = e_ref[...]
    esqh = 0.5 * jnp.sum(e * e, axis=1, keepdims=True)
    for i in range(_BB):
        x = x_ref[i]
        s = jax.lax.dot_general(
            e, x, (((1,), (0,)), ((), ())),
            preferred_element_type=jnp.float32) - esqh
        ind = jnp.argmax(s, axis=0).reshape(1, -1)
        oh = (jax.lax.broadcasted_iota(jnp.int32, s.shape, 0) == ind
              ).astype(jnp.float32)
        zq = jax.lax.dot_general(
            e, oh, (((0,), (0,)), ((), ())),
            preferred_element_type=jnp.float32)
        zq_ref[i] = zq
        ind_ref[i] = ind
        dsum_ref[i] = jnp.sum((zq - x) ** 2).reshape(1, 1)


def kernel(z_e, embed_weight):
    B, C, H, W = z_e.shape
    K = embed_weight.shape[0]
    P = H * W
    x = z_e.reshape(B, C, P)
    zq, ind3, dsums = pl.pallas_call(
        _vq_kernel,
        grid=(B // _BB,),
        in_specs=[
            pl.BlockSpec((_BB, C, P), lambda b: (b, 0, 0)),
            pl.BlockSpec((K, C), lambda b: (0, 0)),
        ],
        out_specs=[
            pl.BlockSpec((_BB, C, P), lambda b: (b, 0, 0)),
            pl.BlockSpec((_BB, 1, P), lambda b: (b, 0, 0)),
            pl.BlockSpec((_BB, 1, 1), lambda b: (b, 0, 0)),
        ],
        out_shape=[
            jax.ShapeDtypeStruct((B, C, P), jnp.float32),
            jax.ShapeDtypeStruct((B, 1, P), jnp.int32),
            jax.ShapeDtypeStruct((B, 1, 1), jnp.float32),
        ],
        compiler_params=pltpu.CompilerParams(
            dimension_semantics=("parallel",)),
    )(x, embed_weight)
    z_q_out = zq.reshape(B, C, H, W)
    ind = ind3.reshape(B, H, W)
    diff = jnp.sum(dsums) * (12.5 / (B * C * H * W))
    return (z_q_out, diff, ind)
